# baseline (speedup 1.0000x reference)
import dataclasses

import jax
import jax.numpy as jnp
from jax import lax
from jax.experimental import pallas as pl
from jax.experimental.pallas import tpu as pltpu
from jax.experimental.pallas import tpu_sc as plsc

_B = 4096
_V = 100000
_NTOP = 1024
_VT = 1000
_NV = _V // _VT

_N_SUBCORES = 32
_CHUNK = _B // _N_SUBCORES
_LANES = 16


def _sc_params():
    cp = pltpu.CompilerParams()
    if "needs_layout_passes" in pltpu.CompilerParams.__dataclass_fields__:
        cp = dataclasses.replace(cp, needs_layout_passes=False)
    return cp


def _sc_mesh():
    return plsc.VectorSubcoreMesh(core_axis_name="c", subcore_axis_name="s")


def _isin_mask(tokens2d, ids, zeros_v):
    @pl.kernel(out_type=jax.ShapeDtypeStruct((1, _B), jnp.float32),
               mesh=_sc_mesh(),
               scratch_types=[pltpu.VMEM((_V,), jnp.int32),
                              pltpu.VMEM((_NTOP,), jnp.int32),
                              pltpu.VMEM((_CHUNK,), jnp.int32),
                              pltpu.VMEM((_CHUNK,), jnp.float32),
                              pltpu.SemaphoreType.DMA,
                              pltpu.SemaphoreType.DMA,
                              pltpu.SemaphoreType.DMA],
               compiler_params=_sc_params())
    def isin_kernel(tokens_hbm, ids_hbm, zeros_hbm, out_hbm, table, ids_v,
                    toks_v, flags_v, sem0, sem1, sem2):
        sub = lax.axis_index("c") * 16 + lax.axis_index("s")
        base = sub * _CHUNK

        ztab = pltpu.async_copy(zeros_hbm, table, sem0)
        cids = pltpu.async_copy(ids_hbm, ids_v, sem1)
        ctok = pltpu.async_copy(tokens_hbm.at[0, pl.ds(base, _CHUNK)],
                                toks_v, sem2)
        ztab.wait()
        cids.wait()

        @pl.loop(0, _NTOP, step=_LANES)
        def _(i):
            plsc.store_scatter(table, [ids_v[pl.ds(i, _LANES)]],
                               jnp.ones((_LANES,), jnp.int32))

        ctok.wait()

        @pl.loop(0, _CHUNK, step=_LANES)
        def _(i):
            fl = plsc.load_gather(table, [toks_v[pl.ds(i, _LANES)]])
            flags_v[pl.ds(i, _LANES)] = fl.astype(jnp.float32)

        pltpu.async_copy(flags_v, out_hbm.at[0, pl.ds(base, _CHUNK)],
                         sem0).wait()

    return isin_kernel(tokens2d, ids, zeros_v)


def _platt_finalize(mask, m, s, xt, params16):
    @pl.kernel(out_type=jax.ShapeDtypeStruct((1, _B), jnp.float32),
               mesh=_sc_mesh(),
               scratch_types=[pltpu.VMEM((_CHUNK,), jnp.float32),
                              pltpu.VMEM((_CHUNK,), jnp.float32),
                              pltpu.VMEM((_CHUNK,), jnp.float32),
                              pltpu.VMEM((_CHUNK,), jnp.float32),
                              pltpu.VMEM((4, _LANES), jnp.float32),
                              pltpu.VMEM((_CHUNK,), jnp.float32),
                              pltpu.SemaphoreType.DMA,
                              pltpu.SemaphoreType.DMA,
                              pltpu.SemaphoreType.DMA,
                              pltpu.SemaphoreType.DMA,
                              pltpu.SemaphoreType.DMA],
               compiler_params=_sc_params())
    def fin_kernel(mask_hbm, m_hbm, s_hbm, xt_hbm, p_hbm, out_hbm,
                   mask_v, m_v, s_v, xt_v, p_v, out_v,
                   sem0, sem1, sem2, sem3, sem4):
        sub = lax.axis_index("c") * 16 + lax.axis_index("s")
        base = sub * _CHUNK
        cols = (0, pl.ds(base, _CHUNK))
        copies = [pltpu.async_copy(mask_hbm.at[*cols], mask_v, sem0),
                  pltpu.async_copy(m_hbm.at[*cols], m_v, sem1),
                  pltpu.async_copy(s_hbm.at[*cols], s_v, sem2),
                  pltpu.async_copy(xt_hbm.at[*cols], xt_v, sem3),
                  pltpu.async_copy(p_hbm, p_v, sem4)]
        for c in copies:
            c.wait()

        @pl.loop(0, _CHUNK, step=_LANES)
        def _(i):
            sl = pl.ds(i, _LANES)
            hit = mask_v[sl] > 0.5
            g = jnp.exp(xt_v[sl] - m_v[sl]) / s_v[sl]
            w = jnp.where(hit, p_v[2, :], p_v[0, :])
            b = jnp.where(hit, p_v[3, :], p_v[1, :])
            z = g * w + b
            out_v[sl] = 1.0 / (1.0 + jnp.exp(-z))

        pltpu.async_copy(out_v, out_hbm.at[*cols], sem0).wait()

    return fin_kernel(mask, m, s, xt, params16)


def _col_kernel(tokens_ref, x_ref, m_ref, s_ref, xt_ref):
    j = pl.program_id(0)

    @pl.when(j == 0)
    def _():
        m_ref[...] = jnp.full((1, _B), -jnp.inf, jnp.float32)
        s_ref[...] = jnp.zeros((1, _B), jnp.float32)
        xt_ref[...] = jnp.zeros((1, _B), jnp.float32)

    tile = x_ref[...]
    tloc = tokens_ref[...] - j * _VT

    loc = jax.lax.broadcasted_iota(jnp.int32, (_VT, _B), 0)
    xt_ref[...] += jnp.sum(jnp.where(loc == tloc, tile, 0.0),
                           axis=0, keepdims=True)

    m_old = m_ref[...]
    m_new = jnp.maximum(m_old, jnp.max(tile, axis=0, keepdims=True))
    s_ref[...] = (s_ref[...] * jnp.exp(m_old - m_new)
                  + jnp.sum(jnp.exp(tile - m_new), axis=0, keepdims=True))
    m_ref[...] = m_new


def kernel(x, tokens, top_token_ids, gen_w, gen_b, top_w, top_b):
    xt_view = x.T
    tokens2d = tokens.astype(jnp.int32).reshape(1, _B)
    ids = top_token_ids.astype(jnp.int32)

    mask = _isin_mask(tokens2d, ids, jnp.zeros((_V,), jnp.int32))

    m, s, xt = pl.pallas_call(
        _col_kernel,
        grid=(_NV,),
        in_specs=[
            pl.BlockSpec((1, _B), lambda j: (0, 0)),
            pl.BlockSpec((_VT, _B), lambda j: (j, 0)),
        ],
        out_specs=[
            pl.BlockSpec((1, _B), lambda j: (0, 0)),
            pl.BlockSpec((1, _B), lambda j: (0, 0)),
            pl.BlockSpec((1, _B), lambda j: (0, 0)),
        ],
        out_shape=[jax.ShapeDtypeStruct((1, _B), jnp.float32)] * 3,
        compiler_params=pltpu.CompilerParams(
            dimension_semantics=("arbitrary",)),
    )(tokens2d, xt_view)

    bc16 = lambda a: jnp.full((_LANES,), a.reshape(()), jnp.float32)
    params16 = jnp.stack([bc16(gen_w), bc16(gen_b), bc16(top_w), bc16(top_b)])
    out = _platt_finalize(mask, m, s, xt, params16)
    return out.reshape(_B)


# speedup vs baseline: 1.0545x; 1.0545x over previous
import dataclasses

import jax
import jax.numpy as jnp
from jax import lax
from jax.experimental import pallas as pl
from jax.experimental.pallas import tpu as pltpu
from jax.experimental.pallas import tpu_sc as plsc

_B = 4096
_V = 100000
_NTOP = 1024
_VT = 1000
_NV = _V // _VT

_N_SUBCORES = 32
_CHUNK = _B // _N_SUBCORES
_LANES = 16


def _sc_params():
    cp = pltpu.CompilerParams()
    if "needs_layout_passes" in pltpu.CompilerParams.__dataclass_fields__:
        cp = dataclasses.replace(cp, needs_layout_passes=False)
    return cp


def _sc_mesh():
    return plsc.VectorSubcoreMesh(core_axis_name="c", subcore_axis_name="s")


def _isin_mask(tokens2d, ids, zeros_v):
    @pl.kernel(out_type=jax.ShapeDtypeStruct((1, _B), jnp.float32),
               mesh=_sc_mesh(),
               scratch_types=[pltpu.VMEM((_V,), jnp.int32),
                              pltpu.VMEM((_NTOP,), jnp.int32),
                              pltpu.VMEM((_CHUNK,), jnp.int32),
                              pltpu.VMEM((_CHUNK,), jnp.float32),
                              pltpu.SemaphoreType.DMA,
                              pltpu.SemaphoreType.DMA,
                              pltpu.SemaphoreType.DMA],
               compiler_params=_sc_params())
    def isin_kernel(tokens_hbm, ids_hbm, zeros_hbm, out_hbm, table, ids_v,
                    toks_v, flags_v, sem0, sem1, sem2):
        sub = lax.axis_index("c") * 16 + lax.axis_index("s")
        base = sub * _CHUNK

        ztab = pltpu.async_copy(zeros_hbm, table, sem0)
        cids = pltpu.async_copy(ids_hbm, ids_v, sem1)
        ctok = pltpu.async_copy(tokens_hbm.at[0, pl.ds(base, _CHUNK)],
                                toks_v, sem2)
        ztab.wait()
        cids.wait()

        @pl.loop(0, _NTOP, step=_LANES)
        def _(i):
            plsc.store_scatter(table, [ids_v[pl.ds(i, _LANES)]],
                               jnp.ones((_LANES,), jnp.int32))

        ctok.wait()

        @pl.loop(0, _CHUNK, step=_LANES)
        def _(i):
            fl = plsc.load_gather(table, [toks_v[pl.ds(i, _LANES)]])
            flags_v[pl.ds(i, _LANES)] = fl.astype(jnp.float32)

        pltpu.async_copy(flags_v, out_hbm.at[0, pl.ds(base, _CHUNK)],
                         sem0).wait()

    return isin_kernel(tokens2d, ids, zeros_v)


def _platt_finalize(mask, m, s, xt, params16):
    @pl.kernel(out_type=jax.ShapeDtypeStruct((1, _B), jnp.float32),
               mesh=_sc_mesh(),
               scratch_types=[pltpu.VMEM((_CHUNK,), jnp.float32),
                              pltpu.VMEM((_CHUNK,), jnp.float32),
                              pltpu.VMEM((_CHUNK,), jnp.float32),
                              pltpu.VMEM((_CHUNK,), jnp.float32),
                              pltpu.VMEM((4, _LANES), jnp.float32),
                              pltpu.VMEM((_CHUNK,), jnp.float32),
                              pltpu.SemaphoreType.DMA,
                              pltpu.SemaphoreType.DMA,
                              pltpu.SemaphoreType.DMA,
                              pltpu.SemaphoreType.DMA,
                              pltpu.SemaphoreType.DMA],
               compiler_params=_sc_params())
    def fin_kernel(mask_hbm, m_hbm, s_hbm, xt_hbm, p_hbm, out_hbm,
                   mask_v, m_v, s_v, xt_v, p_v, out_v,
                   sem0, sem1, sem2, sem3, sem4):
        sub = lax.axis_index("c") * 16 + lax.axis_index("s")
        base = sub * _CHUNK
        cols = (0, pl.ds(base, _CHUNK))
        copies = [pltpu.async_copy(mask_hbm.at[*cols], mask_v, sem0),
                  pltpu.async_copy(m_hbm.at[*cols], m_v, sem1),
                  pltpu.async_copy(s_hbm.at[*cols], s_v, sem2),
                  pltpu.async_copy(xt_hbm.at[*cols], xt_v, sem3),
                  pltpu.async_copy(p_hbm, p_v, sem4)]
        for c in copies:
            c.wait()

        @pl.loop(0, _CHUNK, step=_LANES)
        def _(i):
            sl = pl.ds(i, _LANES)
            hit = mask_v[sl] > 0.5
            g = jnp.exp(xt_v[sl] - m_v[sl]) / s_v[sl]
            w = jnp.where(hit, p_v[2, :], p_v[0, :])
            b = jnp.where(hit, p_v[3, :], p_v[1, :])
            z = g * w + b
            out_v[sl] = 1.0 / (1.0 + jnp.exp(-z))

        pltpu.async_copy(out_v, out_hbm.at[*cols], sem0).wait()

    return fin_kernel(mask, m, s, xt, params16)


def _col_kernel(tokens_ref, x_ref, m_out, s_out, xt_out, m_ref, s_ref,
                xt_ref):
    j = pl.program_id(0)

    @pl.when(j == 0)
    def _():
        m_ref[...] = jnp.full((1, _B), -jnp.inf, jnp.float32)
        s_ref[...] = jnp.zeros((1, _B), jnp.float32)
        xt_ref[...] = jnp.zeros((1, _B), jnp.float32)

    tile = x_ref[...]
    tloc = tokens_ref[...] - j * _VT

    loc = jax.lax.broadcasted_iota(jnp.int32, (_VT, _B), 0)
    xt_ref[...] += jnp.sum(jnp.where(loc == tloc, tile, 0.0),
                           axis=0, keepdims=True)

    m_old = m_ref[...]
    m_new = jnp.maximum(m_old, jnp.max(tile, axis=0, keepdims=True))
    s_ref[...] = (s_ref[...] * jnp.exp(m_old - m_new)
                  + jnp.sum(jnp.exp(tile - m_new), axis=0, keepdims=True))
    m_ref[...] = m_new

    @pl.when(j == _NV - 1)
    def _():
        m_out[...] = m_ref[...]
        s_out[...] = s_ref[...]
        xt_out[...] = xt_ref[...]


def kernel(x, tokens, top_token_ids, gen_w, gen_b, top_w, top_b):
    xt_view = x.T
    tokens2d = tokens.astype(jnp.int32).reshape(1, _B)
    ids = top_token_ids.astype(jnp.int32)

    mask = _isin_mask(tokens2d, ids, jnp.zeros((_V,), jnp.int32))

    m, s, xt = pl.pallas_call(
        _col_kernel,
        grid=(_NV,),
        in_specs=[
            pl.BlockSpec((1, _B), lambda j: (0, 0)),
            pl.BlockSpec((_VT, _B), lambda j: (j, 0)),
        ],
        out_specs=[
            pl.BlockSpec((1, _B), lambda j: (0, 0)),
            pl.BlockSpec((1, _B), lambda j: (0, 0)),
            pl.BlockSpec((1, _B), lambda j: (0, 0)),
        ],
        out_shape=[jax.ShapeDtypeStruct((1, _B), jnp.float32)] * 3,
        scratch_shapes=[pltpu.VMEM((1, _B), jnp.float32)] * 3,
        compiler_params=pltpu.CompilerParams(
            dimension_semantics=("arbitrary",)),
    )(tokens2d, xt_view)

    bc16 = lambda a: jnp.full((_LANES,), a.reshape(()), jnp.float32)
    params16 = jnp.stack([bc16(gen_w), bc16(gen_b), bc16(top_w), bc16(top_b)])
    out = _platt_finalize(mask, m, s, xt, params16)
    return out.reshape(_B)


# speedup vs baseline: 1.1868x; 1.1254x over previous
import dataclasses

import jax
import jax.numpy as jnp
from jax import lax
from jax.experimental import pallas as pl
from jax.experimental.pallas import tpu as pltpu
from jax.experimental.pallas import tpu_sc as plsc

_B = 4096
_V = 100000
_NTOP = 1024
_VT = 1000
_NV = _V // _VT

_N_SUBCORES = 32
_CHUNK = _B // _N_SUBCORES
_LANES = 16


def _sc_params():
    cp = pltpu.CompilerParams()
    if "needs_layout_passes" in pltpu.CompilerParams.__dataclass_fields__:
        cp = dataclasses.replace(cp, needs_layout_passes=False)
    return cp


def _sc_mesh():
    return plsc.VectorSubcoreMesh(core_axis_name="c", subcore_axis_name="s")


def _sc_prep(tokens2d, ids, zeros_v, x_vb):
    @pl.kernel(out_type=[jax.ShapeDtypeStruct((1, _B), jnp.float32),
                         jax.ShapeDtypeStruct((1, _B), jnp.float32)],
               mesh=_sc_mesh(),
               scratch_types=[pltpu.VMEM((_V,), jnp.int32),
                              pltpu.VMEM((_NTOP,), jnp.int32),
                              pltpu.VMEM((_CHUNK,), jnp.int32),
                              pltpu.VMEM((_CHUNK,), jnp.float32),
                              pltpu.VMEM((_CHUNK, 8), jnp.float32),
                              pltpu.VMEM((_CHUNK,), jnp.float32),
                              pltpu.SemaphoreType.DMA,
                              pltpu.SemaphoreType.DMA,
                              pltpu.SemaphoreType.DMA,
                              pltpu.SemaphoreType.DMA],
               compiler_params=_sc_params())
    def prep_kernel(tokens_hbm, ids_hbm, zeros_hbm, x_hbm, mask_hbm,
                    xt_hbm, table, ids_v, toks_v, flags_v, win_v, xt_v,
                    sem0, sem1, sem2, sem3):
        sub = lax.axis_index("c") * 16 + lax.axis_index("s")
        base = sub * _CHUNK

        ztab = pltpu.async_copy(zeros_hbm, table, sem0)
        cids = pltpu.async_copy(ids_hbm, ids_v, sem1)
        ctok = pltpu.async_copy(tokens_hbm.at[0, pl.ds(base, _CHUNK)],
                                toks_v, sem2)
        ctok.wait()

        gathers = []
        for k in range(_CHUNK // _LANES):
            chunk = toks_v[pl.ds(k * _LANES, _LANES)]
            for e in range(_LANES):
                i = k * _LANES + e
                lane0 = base + (i // 8) * 8
                gathers.append(pltpu.async_copy(
                    x_hbm.at[chunk[e], pl.ds(lane0, 8)], win_v.at[i],
                    sem3))

        ztab.wait()
        cids.wait()

        @pl.loop(0, _NTOP, step=_LANES)
        def _(i):
            plsc.store_scatter(table, [ids_v[pl.ds(i, _LANES)]],
                               jnp.ones((_LANES,), jnp.int32))

        @pl.loop(0, _CHUNK, step=_LANES)
        def _(i):
            fl = plsc.load_gather(table, [toks_v[pl.ds(i, _LANES)]])
            flags_v[pl.ds(i, _LANES)] = fl.astype(jnp.float32)

        cmask = pltpu.async_copy(flags_v, mask_hbm.at[0, pl.ds(base, _CHUNK)],
                                 sem0)

        for g in gathers:
            g.wait()

        lane_pat = lax.iota(jnp.int32, _LANES) % 8
        @pl.loop(0, _CHUNK, step=_LANES)
        def _(i):
            rows = i + lax.iota(jnp.int32, _LANES)
            xt_v[pl.ds(i, _LANES)] = plsc.load_gather(win_v, [rows, lane_pat])

        pltpu.async_copy(xt_v, xt_hbm.at[0, pl.ds(base, _CHUNK)],
                         sem1).wait()
        cmask.wait()

    return prep_kernel(tokens2d, ids, zeros_v, x_vb)


def _platt_finalize(mask, m, s, xt, params16):
    @pl.kernel(out_type=jax.ShapeDtypeStruct((1, _B), jnp.float32),
               mesh=_sc_mesh(),
               scratch_types=[pltpu.VMEM((_CHUNK,), jnp.float32),
                              pltpu.VMEM((_CHUNK,), jnp.float32),
                              pltpu.VMEM((_CHUNK,), jnp.float32),
                              pltpu.VMEM((_CHUNK,), jnp.float32),
                              pltpu.VMEM((4, _LANES), jnp.float32),
                              pltpu.VMEM((_CHUNK,), jnp.float32),
                              pltpu.SemaphoreType.DMA,
                              pltpu.SemaphoreType.DMA,
                              pltpu.SemaphoreType.DMA,
                              pltpu.SemaphoreType.DMA,
                              pltpu.SemaphoreType.DMA],
               compiler_params=_sc_params())
    def fin_kernel(mask_hbm, m_hbm, s_hbm, xt_hbm, p_hbm, out_hbm,
                   mask_v, m_v, s_v, xt_v, p_v, out_v,
                   sem0, sem1, sem2, sem3, sem4):
        sub = lax.axis_index("c") * 16 + lax.axis_index("s")
        base = sub * _CHUNK
        cols = (0, pl.ds(base, _CHUNK))
        copies = [pltpu.async_copy(mask_hbm.at[*cols], mask_v, sem0),
                  pltpu.async_copy(m_hbm.at[*cols], m_v, sem1),
                  pltpu.async_copy(s_hbm.at[*cols], s_v, sem2),
                  pltpu.async_copy(xt_hbm.at[*cols], xt_v, sem3),
                  pltpu.async_copy(p_hbm, p_v, sem4)]
        for c in copies:
            c.wait()

        @pl.loop(0, _CHUNK, step=_LANES)
        def _(i):
            sl = pl.ds(i, _LANES)
            hit = mask_v[sl] > 0.5
            g = jnp.exp(xt_v[sl] - m_v[sl]) / s_v[sl]
            w = jnp.where(hit, p_v[2, :], p_v[0, :])
            b = jnp.where(hit, p_v[3, :], p_v[1, :])
            z = g * w + b
            out_v[sl] = 1.0 / (1.0 + jnp.exp(-z))

        pltpu.async_copy(out_v, out_hbm.at[*cols], sem0).wait()

    return fin_kernel(mask, m, s, xt, params16)


def _col_kernel(x_ref, m_out, s_out, m_ref, s_ref):
    j = pl.program_id(0)

    @pl.when(j == 0)
    def _():
        m_ref[...] = jnp.full((1, _B), -jnp.inf, jnp.float32)
        s_ref[...] = jnp.zeros((1, _B), jnp.float32)

    tile = x_ref[...]

    m_old = m_ref[...]
    m_new = jnp.maximum(m_old, jnp.max(tile, axis=0, keepdims=True))
    s_ref[...] = (s_ref[...] * jnp.exp(m_old - m_new)
                  + jnp.sum(jnp.exp(tile - m_new), axis=0, keepdims=True))
    m_ref[...] = m_new

    @pl.when(j == _NV - 1)
    def _():
        m_out[...] = m_ref[...]
        s_out[...] = s_ref[...]


def kernel(x, tokens, top_token_ids, gen_w, gen_b, top_w, top_b):
    xt_view = x.T
    tokens2d = tokens.astype(jnp.int32).reshape(1, _B)
    ids = top_token_ids.astype(jnp.int32)

    mask, xt = _sc_prep(tokens2d, ids, jnp.zeros((_V,), jnp.int32), xt_view)

    m, s = pl.pallas_call(
        _col_kernel,
        grid=(_NV,),
        in_specs=[
            pl.BlockSpec((_VT, _B), lambda j: (j, 0)),
        ],
        out_specs=[
            pl.BlockSpec((1, _B), lambda j: (0, 0)),
            pl.BlockSpec((1, _B), lambda j: (0, 0)),
        ],
        out_shape=[jax.ShapeDtypeStruct((1, _B), jnp.float32)] * 2,
        scratch_shapes=[pltpu.VMEM((1, _B), jnp.float32)] * 2,
        compiler_params=pltpu.CompilerParams(
            dimension_semantics=("arbitrary",)),
    )(xt_view)

    bc16 = lambda a: jnp.full((_LANES,), a.reshape(()), jnp.float32)
    params16 = jnp.stack([bc16(gen_w), bc16(gen_b), bc16(top_w), bc16(top_b)])
    out = _platt_finalize(mask, m, s, xt, params16)
    return out.reshape(_B)
